# baseline (speedup 1.0000x reference)
import functools

import jax
import jax.numpy as jnp
from jax import lax
from jax.experimental import pallas as pl
from jax.experimental.pallas import tpu as pltpu
from jax.experimental.pallas import tpu_sc as plsc

_R = 384
_S = 2048
_D = 128
_BH = 32
_RB = 8


def _row_pos(q, kk):
    c_row = lax.dot_general(q, kk, (((1,), (1,)), ((), ())),
                            preferred_element_type=jnp.float32)
    c_col = lax.transpose(c_row[:, 1:_R + 1], (1, 0))

    base = jnp.sum((c_row < c_col).astype(jnp.int32),
                   axis=1, keepdims=True)
    cL = c_row[:, :_R]
    jT = lax.broadcasted_iota(jnp.int32, (_R, _R), 1)
    iT = lax.broadcasted_iota(jnp.int32, (_R, _R), 0) + 1
    tie = jnp.sum(((cL == c_col) & (jT < iT)).astype(jnp.int32),
                  axis=1, keepdims=True)
    c0 = c_row[:, :1]
    corr = (c0 <= c_col).astype(jnp.int32)
    pos = base + tie - corr + 1
    return lax.transpose(pos, (1, 0))


def _tc_block_kernel(q_ref, k_ref, pos_ref):
    for r in range(_RB):
        pos_ref[r] = _row_pos(q_ref[r], k_ref[r])


def _tc_positions(q, k):
    q0 = q[:, :1, :]
    pos = pl.pallas_call(
        _tc_block_kernel,
        grid=(_BH // _RB,),
        in_specs=[
            pl.BlockSpec((_RB, 1, _D), lambda b: (b, 0, 0)),
            pl.BlockSpec((_RB, _S, _D), lambda b: (b, 0, 0)),
        ],
        out_specs=pl.BlockSpec((_RB, 1, _R), lambda b: (b, 0, 0)),
        out_shape=jax.ShapeDtypeStruct((_BH, 1, _R), jnp.int32),
    )(q0, k)
    return pos.reshape(_BH, _R)


def _sc_scatter(pos):
    mesh = plsc.VectorSubcoreMesh(core_axis_name="c", subcore_axis_name="s")

    @functools.partial(
        pl.kernel,
        mesh=mesh,
        compiler_params=pltpu.CompilerParams(needs_layout_passes=False),
        out_type=jax.ShapeDtypeStruct((_BH, _S), jnp.int32),
        scratch_types=[
            pltpu.VMEM((_R,), jnp.int32),
            pltpu.VMEM((_S,), jnp.int32),
        ],
    )
    def scatter_kernel(pos_hbm, out_hbm, pos_v, row_v):
        wid = lax.axis_index("s") * 2 + lax.axis_index("c")
        pltpu.sync_copy(pos_hbm.at[wid], pos_v)
        zeros16 = jnp.zeros((16,), jnp.int32)
        ones16 = jnp.ones((16,), jnp.int32)

        def zbody(i, carry):
            row_v[pl.ds(i * 16, 16)] = zeros16
            return carry
        lax.fori_loop(0, _S // 16, zbody, 0)
        row_v[pl.ds(0, 16)] = jnp.where(
            lax.iota(jnp.int32, 16) == 0, 1, 0).astype(jnp.int32)

        def sbody(i, carry):
            idx = pos_v[pl.ds(i * 16, 16)]
            plsc.store_scatter(row_v, [idx], ones16)
            return carry
        lax.fori_loop(0, _R // 16, sbody, 0)
        pltpu.sync_copy(row_v, out_hbm.at[wid])

    return scatter_kernel(pos)


def kernel(q, k):
    pos = _tc_positions(q, k)
    return _sc_scatter(pos) != 0


# speedup vs baseline: 1.1847x; 1.1847x over previous
import jax
import jax.numpy as jnp
from jax import lax
from jax.experimental import pallas as pl

_R = 384
_S = 2048
_D = 128
_BH = 32
_RB = 16


def _row_body(q, kk):
    c_row = lax.dot_general(q, kk, (((1,), (1,)), ((), ())),
                            preferred_element_type=jnp.float32)
    c_col = lax.transpose(c_row[:, 1:_R + 1], (1, 0))

    base = jnp.sum((c_row < c_col).astype(jnp.int32),
                   axis=1, keepdims=True)
    cL = c_row[:, :_R]
    jT = lax.broadcasted_iota(jnp.int32, (_R, _R), 1)
    iT = lax.broadcasted_iota(jnp.int32, (_R, _R), 0) + 1
    tie = jnp.sum(((cL == c_col) & (jT < iT)).astype(jnp.int32),
                  axis=1, keepdims=True)
    c0 = c_row[:, :1]
    corr = (c0 <= c_col).astype(jnp.int32)
    pos = base + tie - corr + 1

    j2i = lax.broadcasted_iota(jnp.int32, (_R, _S), 1)
    hit = jnp.any(j2i == pos, axis=0, keepdims=True)
    row0 = lax.broadcasted_iota(jnp.int32, (1, _S), 1) == 0
    return jnp.where(hit | row0, 1, 0).astype(jnp.int32)


def _block_kernel(q_ref, k_ref, out_ref):
    for r in range(_RB):
        out_ref[r] = _row_body(q_ref[r], k_ref[r])


def kernel(q, k):
    q0 = q[:, :1, :]
    mask_i32 = pl.pallas_call(
        _block_kernel,
        grid=(_BH // _RB,),
        in_specs=[
            pl.BlockSpec((_RB, 1, _D), lambda b: (b, 0, 0)),
            pl.BlockSpec((_RB, _S, _D), lambda b: (b, 0, 0)),
        ],
        out_specs=pl.BlockSpec((_RB, 1, _S), lambda b: (b, 0, 0)),
        out_shape=jax.ShapeDtypeStruct((_BH, 1, _S), jnp.int32),
    )(q0, k)
    return mask_i32[:, 0, :] != 0


# speedup vs baseline: 1.2527x; 1.0574x over previous
import jax
import jax.numpy as jnp
from jax import lax
from jax.experimental import pallas as pl

_R = 384
_S = 2048
_D = 128
_BH = 32
_RB = 8


def _row_body(q, kk):
    c_row = lax.dot_general(q, kk, (((1,), (1,)), ((), ())),
                            preferred_element_type=jnp.float32)
    c_col = lax.transpose(c_row[:, 1:_R + 1], (1, 0))

    base = jnp.sum((c_row < c_col).astype(jnp.int32),
                   axis=1, keepdims=True)
    cL = c_row[:, :_R]
    jT = lax.broadcasted_iota(jnp.int32, (_R, _R), 1)
    iT = lax.broadcasted_iota(jnp.int32, (_R, _R), 0) + 1
    tie = jnp.sum(((cL == c_col) & (jT < iT)).astype(jnp.int32),
                  axis=1, keepdims=True)
    c0 = c_row[:, :1]
    corr = (c0 <= c_col).astype(jnp.int32)
    pos = base + tie - corr + 1

    j2i = lax.broadcasted_iota(jnp.int32, (_R, _S), 1)
    hit = jnp.sum((j2i == pos).astype(jnp.int32),
                  axis=0, keepdims=True)
    row0 = (lax.broadcasted_iota(jnp.int32, (1, _S), 1) == 0).astype(jnp.int32)
    return jnp.minimum(hit + row0, 1)


def _block_kernel(q_ref, k_ref, out_ref):
    for r in range(_RB):
        out_ref[r] = _row_body(q_ref[r], k_ref[r])


def kernel(q, k):
    q0 = q[:, :1, :]
    mask_i32 = pl.pallas_call(
        _block_kernel,
        grid=(_BH // _RB,),
        in_specs=[
            pl.BlockSpec((_RB, 1, _D), lambda b: (b, 0, 0)),
            pl.BlockSpec((_RB, _S, _D), lambda b: (b, 0, 0)),
        ],
        out_specs=pl.BlockSpec((_RB, 1, _S), lambda b: (b, 0, 0)),
        out_shape=jax.ShapeDtypeStruct((_BH, 1, _S), jnp.int32),
    )(q0, k)
    return mask_i32[:, 0, :] != 0
